# baseline (speedup 1.0000x reference)
import functools

import jax
import jax.numpy as jnp
from jax import lax
from jax.experimental import pallas as pl
from jax.experimental.pallas import tpu as pltpu
from jax.experimental.pallas import tpu_sc as plsc

_L = 16
_VC = 512


def _mesh():
    return plsc.VectorSubcoreMesh(core_axis_name="c", subcore_axis_name="s")


def _make_detile(F, V, D, NW):
    nfull = V // _VC
    vrem = V - nfull * _VC
    nblocks = F * nfull
    niter = (nblocks + NW - 1) // NW

    @functools.partial(
        pl.kernel,
        mesh=_mesh(),
        compiler_params=pltpu.CompilerParams(
            use_tc_tiling_on_sc=True, needs_layout_passes=False
        ),
        out_type=jax.ShapeDtypeStruct((F * V * D,), jnp.float32),
        scratch_types=[
            pltpu.VMEM((1, D, _VC + 1), jnp.float32),
            pltpu.VMEM((1, D, _VC + 1), jnp.float32),
            pltpu.VMEM((_VC * D,), jnp.float32),
            pltpu.VMEM((_VC * D,), jnp.float32),
            pltpu.SemaphoreType.DMA,
            pltpu.SemaphoreType.DMA,
            pltpu.SemaphoreType.DMA,
        ],
    )
    def detile(wt_hbm, wtail_hbm, wflat_hbm, in_a, in_b, t1_a, t1_b,
               sin0, sin1, sout):
        wid = lax.axis_index("s") * 2 + lax.axis_index("c")
        lane = lax.iota(jnp.int32, _L)
        laneB = lane + _L
        zeros = lane * 0
        sins = (sin0, sin1)
        ins = (in_a, in_b)
        t1s = (t1_a, t1_b)

        def fv(k):
            b_id = k * NW + wid
            return b_id < nblocks, b_id // nfull, (b_id % nfull) * _VC

        def start_in(k, s):
            ok, f, v0 = fv(k)

            @pl.when(ok)
            def _():
                pltpu.async_copy(
                    wt_hbm.at[pl.ds(f, 1), :, pl.ds(v0, _VC)],
                    ins[s].at[:, :, pl.ds(0, _VC)],
                    sins[s],
                )

        def wait_in(s, f, v0):
            pltpu.make_async_copy(
                wt_hbm.at[pl.ds(f, 1), :, pl.ds(v0, _VC)],
                ins[s].at[:, :, pl.ds(0, _VC)],
                sins[s],
            ).wait()

        def wait_out():
            pltpu.make_async_copy(
                wflat_hbm.at[pl.ds(0, _VC * D)], t1_a, sout
            ).wait()

        start_in(0, 0)
        start_in(1, 1)

        def blk_body(j, carry):
            for s in (0, 1):
                k = 2 * j + s
                ok, f, v0 = fv(k)

                @pl.when(ok)
                def _():
                    wait_in(s, f, v0)

                    @plsc.parallel_loop(0, _VC, unroll=4)
                    def _(v):
                        vv = jnp.full((_L,), v, jnp.int32)
                        xa = plsc.load_gather(ins[s], [zeros, lane, vv])
                        xb = plsc.load_gather(ins[s], [zeros, laneB, vv])
                        t1s[s][pl.ds(v * D, _L)] = xa
                        t1s[s][pl.ds(v * D + _L, _L)] = xb

                    @pl.when(k > 0)
                    def _():
                        wait_out()

                    pltpu.async_copy(
                        t1s[s],
                        wflat_hbm.at[pl.ds((f * V + v0) * D, _VC * D)],
                        sout,
                    )
                start_in(k + 2, s)
            return carry

        lax.fori_loop(0, (niter + 1) // 2, blk_body, 0)
        wait_out()

        if vrem:
            @pl.when(wid < F)
            def _():
                t_sl = t1_a.at[pl.ds(0, vrem * D)]
                pltpu.sync_copy(wtail_hbm.at[pl.ds(wid * vrem * D, vrem * D)], t_sl)
                pltpu.sync_copy(
                    t_sl, wflat_hbm.at[pl.ds((wid * V + nfull * _VC) * D, vrem * D)]
                )

    return detile


def _make_emb(B, F, V, D, NW):
    CB = 1024
    nchunks = F * (B // CB)
    cpw = nchunks // NW
    nb = B // CB

    @functools.partial(
        pl.kernel,
        mesh=_mesh(),
        compiler_params=pltpu.CompilerParams(use_tc_tiling_on_sc=False),
        out_type=jax.ShapeDtypeStruct((F, B, D), jnp.float32),
        scratch_types=[
            pltpu.VMEM((2, CB), jnp.int32),
            pltpu.VMEM((2, CB, D), jnp.float32),
            pltpu.SemaphoreType.DMA,
            pltpu.SemaphoreType.DMA,
        ],
    )
    def emb(xt_hbm, table_hbm, out_hbm, idx_v, rows_v, sem0, sem1):
        wid = lax.axis_index("s") * 2 + lax.axis_index("c")
        cid0 = wid * cpw
        sems = (sem0, sem1)

        def fb(k):
            cid = cid0 + k
            return cid // nb, (cid % nb) * CB

        def start(k):
            s = k % 2
            f, b0 = fb(k)
            pltpu.sync_copy(xt_hbm.at[f, pl.ds(b0, CB)], idx_v.at[s])
            return pltpu.async_copy(
                table_hbm.at[f].at[idx_v.at[s]], rows_v.at[s], sems[s]
            )

        pending = start(0)
        for k in range(cpw):
            nxt = start(k + 1) if k + 1 < cpw else None
            pending.wait()
            f, b0 = fb(k)
            pltpu.sync_copy(rows_v.at[k % 2], out_hbm.at[f, pl.ds(b0, CB)])
            pending = nxt

    return emb


def kernel(x, W):
    B, F = x.shape
    _, V, D = W.shape
    info = plsc.get_sparse_core_info()
    NW = info.num_cores * info.num_subcores
    nfull = V // _VC
    wtail = W[:, nfull * _VC:, :].reshape(-1)
    wflat = _make_detile(F, V, D, NW)(W.transpose(0, 2, 1), wtail)
    table = wflat.reshape(F, V, D)
    out = _make_emb(B, F, V, D, NW)(x.T, table)
    return out.transpose(1, 0, 2).reshape(B, F * D)


# speedup vs baseline: 1.1650x; 1.1650x over previous
import functools

import jax
import jax.numpy as jnp
from jax import lax
from jax.experimental import pallas as pl
from jax.experimental.pallas import tpu as pltpu
from jax.experimental.pallas import tpu_sc as plsc

_LANES = 16


def _make_kernel(B, F, V, D, NW):
    BF = B * F
    assert BF % NW == 0
    npw = BF // NW
    assert npw % F == 0
    C = 1024
    assert npw % C == 0
    nchunk = npw // C

    mesh = plsc.VectorSubcoreMesh(core_axis_name="c", subcore_axis_name="s")

    @functools.partial(
        pl.kernel,
        mesh=mesh,
        compiler_params=pltpu.CompilerParams(use_tc_tiling_on_sc=False),
        out_type=jax.ShapeDtypeStruct((BF, D), jnp.float32),
        scratch_types=[
            pltpu.VMEM((npw,), jnp.int32),
            pltpu.VMEM((npw,), jnp.int32),
            pltpu.VMEM((2, C, D), jnp.float32),
            pltpu.SemaphoreType.DMA,
            pltpu.SemaphoreType.DMA,
        ],
    )
    def emb(x_hbm, foffs_hbm, table_hbm, out_hbm, idx_v, foffs_v, rows_v, sem0, sem1):
        wid = lax.axis_index("s") * 2 + lax.axis_index("c")
        base = wid * npw

        pltpu.sync_copy(x_hbm.at[pl.ds(base, npw)], idx_v)
        pltpu.sync_copy(foffs_hbm, foffs_v)

        def add_body(i, carry):
            s = pl.ds(i * _LANES, _LANES)
            idx_v[s] = idx_v[s] + foffs_v[s]
            return carry

        lax.fori_loop(0, npw // _LANES, add_body, 0)

        sems = (sem0, sem1)

        def gather(c):
            return pltpu.async_copy(
                table_hbm.at[idx_v.at[pl.ds(c * C, C)]],
                rows_v.at[c % 2],
                sems[c % 2],
            )

        pending = gather(0)
        for c in range(nchunk):
            nxt = gather(c + 1) if c + 1 < nchunk else None
            pending.wait()
            pltpu.sync_copy(rows_v.at[c % 2], out_hbm.at[pl.ds(base + c * C, C)])
            pending = nxt

    return emb


def kernel(x, W):
    B, F = x.shape
    _, V, D = W.shape
    info = plsc.get_sparse_core_info()
    NW = info.num_cores * info.num_subcores
    x_flat = x.reshape(BF := B * F)
    W_flat = W.reshape(F * V, D)
    npw = BF // NW
    foffs = jnp.tile(jnp.arange(F, dtype=jnp.int32) * V, npw // F)
    out = _make_kernel(B, F, V, D, NW)(x_flat, foffs, W_flat)
    return out.reshape(B, F * D)
